# baseline (speedup 1.0000x reference)
import functools

import jax
import jax.numpy as jnp
from jax.experimental import pallas as pl
from jax.experimental.pallas import tpu as pltpu

_NB = 4


def _se_kernel(x_ref, w1t_ref, b1_ref, w2t_ref, b2_ref, o_hbm,
               obuf, sem, *, Nb, steps_per_core, inv_hw):
    c = pl.program_id(0)
    j = pl.program_id(1)
    slot = jax.lax.rem(j, 2)
    idx0 = (c * steps_per_core + j) * Nb

    def out_copy(b, image_idx, slot_):
        return pltpu.make_async_copy(
            obuf.at[slot_, pl.ds(b, 1)],
            o_hbm.at[pl.ds(image_idx, 1)],
            sem.at[slot_, b])

    @pl.when(j >= 2)
    def _():
        for b in range(Nb):
            out_copy(b, idx0 + b, slot).wait()

    for b in range(Nb):
        x = x_ref[b]
        pooled = jnp.sum(x, axis=1, keepdims=True) * inv_hw
        h = jnp.dot(w1t_ref[...], pooled,
                    preferred_element_type=jnp.float32) + b1_ref[...]
        h = jnp.maximum(h, 0.0)
        y = jax.nn.sigmoid(
            jnp.dot(w2t_ref[...], h,
                    preferred_element_type=jnp.float32) + b2_ref[...])
        obuf[slot, b] = x * y
        out_copy(b, idx0 + b, slot).start()

    @pl.when(j == steps_per_core - 1)
    def _():
        for b in range(Nb):
            out_copy(b, idx0 + b, slot).wait()
        if steps_per_core > 1:
            for b in range(Nb):
                out_copy(b, idx0 + b, 1 - slot).wait()


def kernel(x, w1, b1, w2, b2):
    N, C, H, W = x.shape
    Cr = w1.shape[1]
    HW = H * W

    x_flat = x.reshape(N, C, HW)
    w1t = w1.T
    w2t = w2.T
    b1c = b1.reshape(Cr, 1)
    b2c = b2.reshape(C, 1)

    Nb = _NB if N % (2 * _NB) == 0 else 1
    cores = 2 if N % 2 == 0 else 1
    steps_per_core = N // (cores * Nb)

    out_flat = pl.pallas_call(
        functools.partial(_se_kernel,
                          Nb=Nb, steps_per_core=steps_per_core,
                          inv_hw=1.0 / float(HW)),
        out_shape=jax.ShapeDtypeStruct((N, C, HW), x.dtype),
        grid=(cores, steps_per_core),
        in_specs=[
            pl.BlockSpec((Nb, C, HW),
                         lambda c, j, spc=steps_per_core: (c * spc + j, 0, 0)),
            pl.BlockSpec((Cr, C), lambda c, j: (0, 0)),
            pl.BlockSpec((Cr, 1), lambda c, j: (0, 0)),
            pl.BlockSpec((C, Cr), lambda c, j: (0, 0)),
            pl.BlockSpec((C, 1), lambda c, j: (0, 0)),
        ],
        out_specs=pl.BlockSpec(memory_space=pltpu.MemorySpace.HBM),
        scratch_shapes=[
            pltpu.VMEM((2, Nb, C, HW), jnp.float32),
            pltpu.SemaphoreType.DMA((2, Nb)),
        ],
        compiler_params=pltpu.CompilerParams(
            dimension_semantics=("parallel", "arbitrary"),
            vmem_limit_bytes=64 << 20,
        ),
    )(x_flat, w1t, b1c, w2t, b2c)

    return out_flat.reshape(N, C, H, W)


# speedup vs baseline: 1.0091x; 1.0091x over previous
import functools

import jax
import jax.numpy as jnp
from jax.experimental import pallas as pl
from jax.experimental.pallas import tpu as pltpu

_NB = 4


def _se_hybrid_kernel(x_ref, w1_ref, b1_ref, w2_ref, b2_ref, o_hbm,
                      obuf, sem, *, Nb, steps_per_core, inv_hw):
    c = pl.program_id(0)
    j = pl.program_id(1)
    slot = jax.lax.rem(j, 2)
    idx0 = (c * steps_per_core + j) * Nb

    def out_copy(b, image_idx, slot_):
        return pltpu.make_async_copy(
            obuf.at[slot_, pl.ds(b, 1)],
            o_hbm.at[pl.ds(image_idx, 1)],
            sem.at[slot_, b])

    @pl.when(j >= 2)
    def _():
        for b in range(Nb):
            out_copy(b, idx0 + b, slot).wait()

    x = x_ref[...]
    pooled = jnp.sum(x, axis=2) * inv_hw
    h = jnp.dot(pooled, w1_ref[...],
                preferred_element_type=jnp.float32) + b1_ref[...]
    h = jnp.maximum(h, 0.0)
    y = jax.nn.sigmoid(
        jnp.dot(h, w2_ref[...],
                preferred_element_type=jnp.float32) + b2_ref[...])

    for b in range(Nb):
        obuf[slot, b] = x[b] * y[b, :, None]
        out_copy(b, idx0 + b, slot).start()

    @pl.when(j == steps_per_core - 1)
    def _():
        for b in range(Nb):
            out_copy(b, idx0 + b, slot).wait()
        if steps_per_core > 1:
            for b in range(Nb):
                out_copy(b, idx0 + b, 1 - slot).wait()


def kernel(x, w1, b1, w2, b2):
    N, C, H, W = x.shape
    Cr = w1.shape[1]
    HW = H * W

    x_flat = x.reshape(N, C, HW)
    b1r = b1.reshape(1, Cr)
    b2r = b2.reshape(1, C)

    Nb = _NB if N % (2 * _NB) == 0 else 1
    cores = 2 if N % 2 == 0 else 1
    steps_per_core = N // (cores * Nb)

    out_flat = pl.pallas_call(
        functools.partial(_se_hybrid_kernel,
                          Nb=Nb, steps_per_core=steps_per_core,
                          inv_hw=1.0 / float(HW)),
        out_shape=jax.ShapeDtypeStruct((N, C, HW), x.dtype),
        grid=(cores, steps_per_core),
        in_specs=[
            pl.BlockSpec((Nb, C, HW),
                         lambda c, j, spc=steps_per_core: (c * spc + j, 0, 0)),
            pl.BlockSpec((C, Cr), lambda c, j: (0, 0)),
            pl.BlockSpec((1, Cr), lambda c, j: (0, 0)),
            pl.BlockSpec((Cr, C), lambda c, j: (0, 0)),
            pl.BlockSpec((1, C), lambda c, j: (0, 0)),
        ],
        out_specs=pl.BlockSpec(memory_space=pltpu.MemorySpace.HBM),
        scratch_shapes=[
            pltpu.VMEM((2, Nb, C, HW), jnp.float32),
            pltpu.SemaphoreType.DMA((2, Nb)),
        ],
        compiler_params=pltpu.CompilerParams(
            dimension_semantics=("parallel", "arbitrary"),
            vmem_limit_bytes=64 << 20,
        ),
    )(x_flat, w1, b1r, w2, b2r)

    return out_flat.reshape(N, C, H, W)


# speedup vs baseline: 1.0118x; 1.0026x over previous
import functools

import jax
import jax.numpy as jnp
from jax.experimental import pallas as pl
from jax.experimental.pallas import tpu as pltpu

_NB = 4


def _se_hybrid_kernel(x_ref, w1_ref, b1_ref, w2_ref, b2_ref, o_hbm,
                      obuf, sem, *, Nb, steps_per_core, inv_hw):
    c = pl.program_id(0)
    j = pl.program_id(1)
    slot = jax.lax.rem(j, 2)
    idx0 = (c * steps_per_core + j) * Nb

    def out_copy(b, image_idx, slot_):
        return pltpu.make_async_copy(
            obuf.at[slot_, pl.ds(b, 1)],
            o_hbm.at[pl.ds(image_idx, 1)],
            sem.at[slot_, b])

    @pl.when(j >= 2)
    def _():
        for b in range(Nb):
            out_copy(b, idx0 + b, slot).wait()

    x0 = x_ref[0:1]
    p0 = jnp.sum(x0, axis=2) * inv_hw
    h0 = jnp.maximum(
        jnp.dot(p0, w1_ref[...],
                preferred_element_type=jnp.float32) + b1_ref[...], 0.0)
    y0 = jax.nn.sigmoid(
        jnp.dot(h0, w2_ref[...],
                preferred_element_type=jnp.float32) + b2_ref[...])
    obuf[slot, 0] = x0[0] * y0[0, :, None]
    out_copy(0, idx0, slot).start()

    if Nb > 1:
        xr = x_ref[1:Nb]
        pooled = jnp.sum(xr, axis=2) * inv_hw
        h = jnp.maximum(
            jnp.dot(pooled, w1_ref[...],
                    preferred_element_type=jnp.float32) + b1_ref[...], 0.0)
        y = jax.nn.sigmoid(
            jnp.dot(h, w2_ref[...],
                    preferred_element_type=jnp.float32) + b2_ref[...])
        for b in range(1, Nb):
            obuf[slot, b] = xr[b - 1] * y[b - 1, :, None]
            out_copy(b, idx0 + b, slot).start()

    @pl.when(j == steps_per_core - 1)
    def _():
        for b in range(Nb):
            out_copy(b, idx0 + b, slot).wait()
        if steps_per_core > 1:
            for b in range(Nb):
                out_copy(b, idx0 + b, 1 - slot).wait()


def kernel(x, w1, b1, w2, b2):
    N, C, H, W = x.shape
    Cr = w1.shape[1]
    HW = H * W

    x_flat = x.reshape(N, C, HW)
    b1r = b1.reshape(1, Cr)
    b2r = b2.reshape(1, C)

    Nb = _NB if N % (2 * _NB) == 0 else 1
    cores = 2 if N % 2 == 0 else 1
    steps_per_core = N // (cores * Nb)

    out_flat = pl.pallas_call(
        functools.partial(_se_hybrid_kernel,
                          Nb=Nb, steps_per_core=steps_per_core,
                          inv_hw=1.0 / float(HW)),
        out_shape=jax.ShapeDtypeStruct((N, C, HW), x.dtype),
        grid=(cores, steps_per_core),
        in_specs=[
            pl.BlockSpec((Nb, C, HW),
                         lambda c, j, spc=steps_per_core: (c * spc + j, 0, 0)),
            pl.BlockSpec((C, Cr), lambda c, j: (0, 0)),
            pl.BlockSpec((1, Cr), lambda c, j: (0, 0)),
            pl.BlockSpec((Cr, C), lambda c, j: (0, 0)),
            pl.BlockSpec((1, C), lambda c, j: (0, 0)),
        ],
        out_specs=pl.BlockSpec(memory_space=pltpu.MemorySpace.HBM),
        scratch_shapes=[
            pltpu.VMEM((2, Nb, C, HW), jnp.float32),
            pltpu.SemaphoreType.DMA((2, Nb)),
        ],
        compiler_params=pltpu.CompilerParams(
            dimension_semantics=("parallel", "arbitrary"),
            vmem_limit_bytes=64 << 20,
        ),
    )(x_flat, w1, b1r, w2, b2r)

    return out_flat.reshape(N, C, H, W)
